# baseline (speedup 1.0000x reference)
import jax
import jax.numpy as jnp
from jax.experimental import pallas as pl

HIDDEN_DIM = 2048
NUM_EXPERTS = 16
HALF_BLOCK = 256
TOKEN_BLOCK = 2 * HALF_BLOCK


def _logits_t(x, cn):
    sumsq = jnp.sum(x * x, axis=1, keepdims=True)
    norm = jnp.maximum(jnp.sqrt(sumsq), 1e-12)
    xn = x / norm
    return jax.lax.dot_general(
        cn, xn, (((1,), (1,)), ((), ())),
        preferred_element_type=jnp.float32,
    )


def _router_block(xa_ref, xb_ref, c_ref, ids_ref, w_ref):
    c = c_ref[:]
    c_sumsq = jnp.sum(c * c, axis=1, keepdims=True)
    cn = c / jnp.maximum(jnp.sqrt(c_sumsq), 1e-12)

    lt = jnp.concatenate(
        [_logits_t(xa_ref[:], cn), _logits_t(xb_ref[:], cn)], axis=1
    )

    iota = jax.lax.broadcasted_iota(jnp.int32, lt.shape, 0)
    m1 = jnp.max(lt, axis=0, keepdims=True)
    id1 = jnp.min(jnp.where(lt == m1, iota, NUM_EXPERTS), axis=0, keepdims=True)
    masked = jnp.where(iota == id1, -1e30, lt)
    m2 = jnp.max(masked, axis=0, keepdims=True)
    id2 = jnp.min(jnp.where(masked == m2, iota, NUM_EXPERTS), axis=0, keepdims=True)

    z = jnp.sum(jnp.exp(lt - m1), axis=0, keepdims=True)
    e2 = jnp.exp(m2 - m1)
    denom = 1.0 + e2 + 1e-9 * z
    w1 = 1.0 / denom
    w2 = e2 / denom

    ids_ref[:] = jnp.concatenate([id1, id2], axis=0)
    w_ref[:] = jnp.concatenate([w1, w2], axis=0)


def kernel(hidden_states, centers):
    B, T, H = hidden_states.shape
    n_tokens = B * T
    flat = hidden_states.reshape(n_tokens, H)
    grid = (n_tokens // TOKEN_BLOCK,)
    ids_t, w_t = pl.pallas_call(
        _router_block,
        grid=grid,
        in_specs=[
            pl.BlockSpec((HALF_BLOCK, H), lambda i: (2 * i, 0)),
            pl.BlockSpec((HALF_BLOCK, H), lambda i: (2 * i + 1, 0)),
            pl.BlockSpec((NUM_EXPERTS, H), lambda i: (0, 0)),
        ],
        out_specs=[
            pl.BlockSpec((2, TOKEN_BLOCK), lambda i: (0, i)),
            pl.BlockSpec((2, TOKEN_BLOCK), lambda i: (0, i)),
        ],
        out_shape=[
            jax.ShapeDtypeStruct((2, n_tokens), jnp.int32),
            jax.ShapeDtypeStruct((2, n_tokens), jnp.float32),
        ],
    )(flat, flat, centers)
    return (
        ids_t.T.reshape(B, T, 2),
        w_t.T.reshape(B, T, 2),
    )


# speedup vs baseline: 1.1587x; 1.1587x over previous
import jax
import jax.numpy as jnp
from jax.experimental import pallas as pl

HIDDEN_DIM = 2048
NUM_EXPERTS = 16
TOKEN_BLOCK = 1024


def _router_block(x_ref, c_ref, ids_ref, w_ref):
    c = c_ref[:]
    c_sumsq = jnp.sum(c * c, axis=1, keepdims=True)
    cn = c / jnp.maximum(jnp.sqrt(c_sumsq), 1e-12)

    x = x_ref[:]
    sumsq = jnp.sum(x * x, axis=1, keepdims=True)
    norm = jnp.maximum(jnp.sqrt(sumsq), 1e-12)
    xn = x / norm
    lt = jax.lax.dot_general(
        cn, xn, (((1,), (1,)), ((), ())),
        preferred_element_type=jnp.float32,
    )

    iota = jax.lax.broadcasted_iota(jnp.int32, lt.shape, 0)
    m1 = jnp.max(lt, axis=0, keepdims=True)
    id1 = jnp.min(jnp.where(lt == m1, iota, NUM_EXPERTS), axis=0, keepdims=True)
    masked = jnp.where(iota == id1, -1e30, lt)
    m2 = jnp.max(masked, axis=0, keepdims=True)
    id2 = jnp.min(jnp.where(masked == m2, iota, NUM_EXPERTS), axis=0, keepdims=True)

    z = jnp.sum(jnp.exp(lt - m1), axis=0, keepdims=True)
    e2 = jnp.exp(m2 - m1)
    denom = 1.0 + e2 + 1e-9 * z
    w1 = 1.0 / denom
    w2 = e2 / denom

    ids_ref[:] = jnp.concatenate([id1, id2], axis=0)
    w_ref[:] = jnp.concatenate([w1, w2], axis=0)


def kernel(hidden_states, centers):
    B, T, H = hidden_states.shape
    n_tokens = B * T
    flat = hidden_states.reshape(n_tokens, H)
    grid = (n_tokens // TOKEN_BLOCK,)
    ids_t, w_t = pl.pallas_call(
        _router_block,
        grid=grid,
        in_specs=[
            pl.BlockSpec((TOKEN_BLOCK, H), lambda i: (i, 0)),
            pl.BlockSpec((NUM_EXPERTS, H), lambda i: (0, 0)),
        ],
        out_specs=[
            pl.BlockSpec((2, TOKEN_BLOCK), lambda i: (0, i)),
            pl.BlockSpec((2, TOKEN_BLOCK), lambda i: (0, i)),
        ],
        out_shape=[
            jax.ShapeDtypeStruct((2, n_tokens), jnp.int32),
            jax.ShapeDtypeStruct((2, n_tokens), jnp.float32),
        ],
    )(flat, centers)
    return (
        ids_t.T.reshape(B, T, 2),
        w_t.T.reshape(B, T, 2),
    )


# speedup vs baseline: 1.2006x; 1.0362x over previous
import jax
import jax.numpy as jnp
from jax.experimental import pallas as pl

HIDDEN_DIM = 2048
NUM_EXPERTS = 16
TOKEN_BLOCK = 2048


def _router_block(x_ref, c_ref, ids_ref, w_ref):
    c = c_ref[:]
    c_sumsq = jnp.sum(c * c, axis=1, keepdims=True)
    cn = c / jnp.maximum(jnp.sqrt(c_sumsq), 1e-12)

    x = x_ref[:]
    sumsq = jnp.sum(x * x, axis=1, keepdims=True)
    norm = jnp.maximum(jnp.sqrt(sumsq), 1e-12)
    xn = x / norm
    lt = jax.lax.dot_general(
        cn, xn, (((1,), (1,)), ((), ())),
        preferred_element_type=jnp.float32,
    )

    iota = jax.lax.broadcasted_iota(jnp.int32, lt.shape, 0)
    m1 = jnp.max(lt, axis=0, keepdims=True)
    id1 = jnp.min(jnp.where(lt == m1, iota, NUM_EXPERTS), axis=0, keepdims=True)
    masked = jnp.where(iota == id1, -1e30, lt)
    m2 = jnp.max(masked, axis=0, keepdims=True)
    id2 = jnp.min(jnp.where(masked == m2, iota, NUM_EXPERTS), axis=0, keepdims=True)

    z = jnp.sum(jnp.exp(lt - m1), axis=0, keepdims=True)
    e2 = jnp.exp(m2 - m1)
    denom = 1.0 + e2 + 1e-9 * z
    w1 = 1.0 / denom
    w2 = e2 / denom

    ids_ref[:] = jnp.concatenate([id1, id2], axis=0)
    w_ref[:] = jnp.concatenate([w1, w2], axis=0)


def kernel(hidden_states, centers):
    B, T, H = hidden_states.shape
    n_tokens = B * T
    flat = hidden_states.reshape(n_tokens, H)
    grid = (n_tokens // TOKEN_BLOCK,)
    ids_t, w_t = pl.pallas_call(
        _router_block,
        grid=grid,
        in_specs=[
            pl.BlockSpec((TOKEN_BLOCK, H), lambda i: (i, 0)),
            pl.BlockSpec((NUM_EXPERTS, H), lambda i: (0, 0)),
        ],
        out_specs=[
            pl.BlockSpec((2, TOKEN_BLOCK), lambda i: (0, i)),
            pl.BlockSpec((2, TOKEN_BLOCK), lambda i: (0, i)),
        ],
        out_shape=[
            jax.ShapeDtypeStruct((2, n_tokens), jnp.int32),
            jax.ShapeDtypeStruct((2, n_tokens), jnp.float32),
        ],
    )(flat, centers)
    return (
        ids_t.T.reshape(B, T, 2),
        w_t.T.reshape(B, T, 2),
    )
